# baseline (speedup 1.0000x reference)
import functools

import jax
import jax.numpy as jnp
from jax import lax
from jax.experimental import pallas as pl
from jax.experimental.pallas import tpu as pltpu
from jax.experimental.pallas import tpu_sc as plsc

_V = 8192
_D = 8192
_B = 4096
_NW = 32
_BPW = _B // _NW
_RPC = 4
_NBUF = 3
_CPW = _BPW // _RPC
_QD = _D // 4

_mesh = plsc.VectorSubcoreMesh(core_axis_name="c", subcore_axis_name="s")


@functools.partial(
    pl.kernel,
    mesh=_mesh,
    out_type=jax.ShapeDtypeStruct((_B * 4, _QD), jnp.float32),
    scratch_types=[
        pltpu.VMEM((_BPW,), jnp.int32),
    ] + [pltpu.VMEM((16, _QD), jnp.float32)] * _NBUF
      + [pltpu.SemaphoreType.DMA] * (2 * _NBUF),
)
def _sc_gather(table_hbm, idx_hbm, out_hbm, idx_v, *bufs_and_sems):
    bufs = bufs_and_sems[:_NBUF]
    gsems = bufs_and_sems[_NBUF:2 * _NBUF]
    ssems = bufs_and_sems[2 * _NBUF:]
    wid = lax.axis_index("s") * 2 + lax.axis_index("c")
    pltpu.sync_copy(idx_hbm.at[wid], idx_v)
    qbase = wid * _BPW * 4
    lane = lax.iota(jnp.int32, 16)

    def qidx(c):
        grp = idx_v[pl.ds((c // 4) * 16, 16)]
        rows = grp.at[(c % 4) * 4 + (lane >> 2)].get(mode="promise_in_bounds")
        return rows * 4 + (lane & 3)

    def out_q(c):
        return out_hbm.at[pl.ds(qbase + c * 16, 16)]

    for j in range(_NBUF):
        pltpu.async_copy(table_hbm.at[qidx(j)], bufs[j], gsems[j])

    n_iter = -(-_CPW // _NBUF)

    def body(i, _):
        c0 = i * _NBUF
        for j in range(_NBUF):
            @pl.when(c0 + j < _CPW)
            def _(j=j):
                pltpu.make_async_copy(
                    table_hbm.at[qidx(c0 + j)], bufs[j], gsems[j]).wait()
                pltpu.async_copy(bufs[j], out_q(c0 + j), ssems[j])

        for j in range(_NBUF):
            @pl.when(c0 + j + _NBUF < _CPW)
            def _(j=j):
                pltpu.make_async_copy(
                    bufs[j], out_q(c0 + j), ssems[j]).wait()
                pltpu.async_copy(
                    table_hbm.at[qidx(c0 + _NBUF + j)], bufs[j], gsems[j])

        return 0

    lax.fori_loop(0, n_iter, body, 0)

    for c in range(max(_CPW - _NBUF, 0), _CPW):
        pltpu.make_async_copy(bufs[c % _NBUF], out_q(c),
                              ssems[c % _NBUF]).wait()


def kernel(idx, table):
    idx2 = idx.reshape(_NW, _BPW).astype(jnp.int32)
    tq = table.reshape(_V * 4, _QD)
    out = _sc_gather(tq, idx2)
    return out.reshape(idx.shape[0], idx.shape[1], _D)


# speedup vs baseline: 4.8052x; 4.8052x over previous
import functools

import jax
import jax.numpy as jnp
from jax import lax
from jax.experimental import pallas as pl
from jax.experimental.pallas import tpu as pltpu
from jax.experimental.pallas import tpu_sc as plsc

_V = 8192
_D = 8192
_B = 4096
_NW = 32
_R = 2
_NBUF = 6
_CPW = (_B // _NW) // _R

_mesh = plsc.VectorSubcoreMesh(core_axis_name="c", subcore_axis_name="s")


@functools.partial(
    pl.kernel,
    mesh=_mesh,
    out_type=jax.ShapeDtypeStruct((_B, _D), jnp.float32),
    scratch_types=[
        pltpu.VMEM((_CPW, _R), jnp.int32),
    ] + [pltpu.VMEM((_R, _D), jnp.float32)] * _NBUF
      + [pltpu.SemaphoreType.DMA] * (2 * _NBUF),
)
def _sc_gather(table_hbm, idx_hbm, out_hbm, idx_v, *bufs_and_sems):
    bufs = bufs_and_sems[:_NBUF]
    gsems = bufs_and_sems[_NBUF:2 * _NBUF]
    ssems = bufs_and_sems[2 * _NBUF:]
    wid = lax.axis_index("s") * 2 + lax.axis_index("c")
    pltpu.sync_copy(idx_hbm.at[wid], idx_v)
    rbase = wid * _CPW * _R

    def out_rows(c):
        return out_hbm.at[pl.ds(rbase + c * _R, _R)]

    for j in range(_NBUF):
        pltpu.async_copy(table_hbm.at[idx_v.at[j]], bufs[j], gsems[j])

    n_iter = -(-_CPW // _NBUF)

    def body(i, _):
        c0 = i * _NBUF
        for j in range(_NBUF):
            @pl.when(c0 + j < _CPW)
            def _(j=j):
                pltpu.make_async_copy(
                    table_hbm.at[idx_v.at[c0 + j]], bufs[j], gsems[j]).wait()
                pltpu.async_copy(bufs[j], out_rows(c0 + j), ssems[j])

        for j in range(_NBUF):
            @pl.when(c0 + j + _NBUF < _CPW)
            def _(j=j):
                pltpu.make_async_copy(
                    bufs[j], out_rows(c0 + j), ssems[j]).wait()
                pltpu.async_copy(
                    table_hbm.at[idx_v.at[c0 + _NBUF + j]], bufs[j], gsems[j])

        return 0

    lax.fori_loop(0, n_iter, body, 0)

    for c in range(max(_CPW - _NBUF, 0), _CPW):
        pltpu.make_async_copy(bufs[c % _NBUF], out_rows(c),
                              ssems[c % _NBUF]).wait()


def kernel(idx, table):
    idx3 = idx.reshape(_NW, _CPW, _R).astype(jnp.int32)
    out = _sc_gather(table, idx3)
    return out.reshape(idx.shape[0], idx.shape[1], _D)
